# baseline (speedup 1.0000x reference)
import jax
import jax.numpy as jnp
from jax.experimental import pallas as pl
from jax.experimental.pallas import tpu as pltpu

B, S, DIM = 1, 2048, 1024
DH, H = 64, 16
E, HID = 8, 4096
EPS = 1e-6
SCALE = DH ** (-0.5)

TQ = 256
TQA = 256
TKA = 256
NQ = S // TQA
NK = S // TKA
VP = 128
KH = 1024
KHS = KH // 2
HB = HID // KH
GPAD = 128

F32 = jnp.float32
BF16 = jnp.bfloat16


def _bdot(a, b):
    return jax.lax.dot_general(a.astype(BF16), b.astype(BF16),
                               (((1,), (0,)), ((), ())),
                               preferred_element_type=F32)


def _qkv_body(x_ref, wq_ref, wk_ref, wv_ref, gq_ref, gk_ref, q_ref, k_ref, v_ref):
    xb = x_ref[...].astype(BF16)
    q = jnp.dot(xb, wq_ref[...], preferred_element_type=F32)
    k = jnp.dot(xb, wk_ref[...], preferred_element_type=F32)
    v = jnp.dot(xb, wv_ref[...], preferred_element_type=F32)
    gq = gq_ref[...]
    gk = gk_ref[...]
    ones_col = (jax.lax.broadcasted_iota(jnp.int32, (TQ, VP - DH), 1) == 0)
    pad = jnp.where(ones_col, 1.0, 0.0).astype(BF16)
    for hh in range(H):
        sl = slice(hh * DH, (hh + 1) * DH)
        qh = q[:, sl]
        kh = k[:, sl]
        qms = jnp.mean(qh * qh, axis=1, keepdims=True)
        kms = jnp.mean(kh * kh, axis=1, keepdims=True)
        q_ref[hh] = (qh * (jax.lax.rsqrt(qms + EPS) * SCALE) * gq[:, sl]).astype(BF16)
        k_ref[hh] = (kh * jax.lax.rsqrt(kms + EPS) * gk[:, sl]).astype(BF16)
        v_ref[hh] = jnp.concatenate([v[:, sl].astype(BF16), pad], axis=1)


def _attn_body(q_ref, k_ref, v_ref, o_ref, acc_ref):
    qi = pl.program_id(1)
    q = q_ref[0]
    acc_ref[...] = jnp.zeros_like(acc_ref)
    rows = qi * TQA + jax.lax.broadcasted_iota(jnp.int32, (TQA, TKA), 0)
    cols0 = jax.lax.broadcasted_iota(jnp.int32, (TQA, TKA), 1)

    for j in range(NK):
        @pl.when(j <= qi)
        def _chunk(j=j):
            kc = k_ref[0, j * TKA:(j + 1) * TKA, :]
            vc = v_ref[0, j * TKA:(j + 1) * TKA, :]
            s = jax.lax.dot_general(q, kc, (((1,), (1,)), ((), ())),
                                    preferred_element_type=F32)
            p = jnp.exp(s)
            p = jnp.where(rows >= j * TKA + cols0, p, 0.0)
            acc_ref[...] += jnp.dot(p.astype(BF16), vc,
                                    preferred_element_type=F32)

    acc = acc_ref[...]
    cols = jax.lax.broadcasted_iota(jnp.int32, acc.shape, 1)
    denom = jnp.sum(jnp.where(cols == DH, acc, 0.0), axis=1, keepdims=True)
    o_ref[0] = (acc[:, :DH] / denom).astype(BF16)


def _moe_body(a_ref, wo_ref, wg_ref, w1_ref, b1_ref, w2_ref, b2_ref,
              out_ref, o_sc, g_sc):
    e = pl.program_id(0)
    h = pl.program_id(1)

    @pl.when((e == 0) & (h == 0))
    def _prologue():
        ob = jnp.zeros((S, DIM), F32)
        for hh in range(H):
            ob = ob + jnp.dot(a_ref[hh], wo_ref[hh * DH:(hh + 1) * DH, :],
                              preferred_element_type=F32)
        o_sc[...] = ob.astype(BF16)
        gl = jnp.dot(o_sc[...], wg_ref[...], preferred_element_type=F32)
        cols = jax.lax.broadcasted_iota(jnp.int32, gl.shape, 1)
        gl = jnp.where(cols < E, gl, -jnp.inf)
        m = jnp.max(gl, axis=1, keepdims=True)
        p = jnp.exp(gl - m)
        g_sc[...] = p / jnp.sum(p, axis=1, keepdims=True)
        out_ref[...] = jnp.zeros_like(out_ref)

    ob = o_sc[...]
    hb0 = _bdot(ob, w1_ref[0, :, :KHS]) + b1_ref[0, :, :KHS]
    hb1 = _bdot(ob, w1_ref[0, :, KHS:]) + b1_ref[0, :, KHS:]
    contrib = (_bdot(jax.nn.gelu(hb0), w2_ref[0, :KHS, :])
               + _bdot(jax.nn.gelu(hb1), w2_ref[0, KHS:, :]))
    g = g_sc[...]
    cols = jax.lax.broadcasted_iota(jnp.int32, g.shape, 1)
    ge = jnp.sum(jnp.where(cols == e, g, 0.0), axis=1, keepdims=True)
    acc = out_ref[...] + ge * contrib

    @pl.when(h == HB - 1)
    def _bias():
        out_ref[...] = acc + ge * b2_ref[0]

    @pl.when(h != HB - 1)
    def _noacc():
        out_ref[...] = acc


def kernel(x, Wq, Wk, Wv, Wo, gq, gk, Wg, W1, b1, W2, b2):
    xs = x.reshape(S, DIM)
    gq_t = jnp.tile(gq, H).reshape(1, H * DH)
    gk_t = jnp.tile(gk, H).reshape(1, H * DH)
    wg_pad = jnp.zeros((DIM, GPAD), BF16).at[:, :E].set(Wg.astype(BF16))
    b1_3d = b1.reshape(E, 1, HID)
    b2_3d = b2.reshape(E, 1, DIM)
    wq_b, wk_b, wv_b = Wq.astype(BF16), Wk.astype(BF16), Wv.astype(BF16)
    wo_b = Wo.astype(BF16)

    q, k, v = pl.pallas_call(
        _qkv_body,
        grid=(S // TQ,),
        in_specs=[
            pl.BlockSpec((TQ, DIM), lambda i: (i, 0)),
            pl.BlockSpec((DIM, H * DH), lambda i: (0, 0)),
            pl.BlockSpec((DIM, H * DH), lambda i: (0, 0)),
            pl.BlockSpec((DIM, H * DH), lambda i: (0, 0)),
            pl.BlockSpec((1, H * DH), lambda i: (0, 0)),
            pl.BlockSpec((1, H * DH), lambda i: (0, 0)),
        ],
        out_specs=[
            pl.BlockSpec((H, TQ, DH), lambda i: (0, i, 0)),
            pl.BlockSpec((H, TQ, DH), lambda i: (0, i, 0)),
            pl.BlockSpec((H, TQ, VP), lambda i: (0, i, 0)),
        ],
        out_shape=[
            jax.ShapeDtypeStruct((H, S, DH), BF16),
            jax.ShapeDtypeStruct((H, S, DH), BF16),
            jax.ShapeDtypeStruct((H, S, VP), BF16),
        ],
    )(xs, wq_b, wk_b, wv_b, gq_t, gk_t)

    attn = pl.pallas_call(
        _attn_body,
        grid=(H, NQ),
        in_specs=[
            pl.BlockSpec((1, TQA, DH), lambda hh, i: (hh, i, 0)),
            pl.BlockSpec((1, S, DH), lambda hh, i: (hh, 0, 0)),
            pl.BlockSpec((1, S, VP), lambda hh, i: (hh, 0, 0)),
        ],
        out_specs=pl.BlockSpec((1, TQA, DH), lambda hh, i: (hh, i, 0)),
        out_shape=jax.ShapeDtypeStruct((H, S, DH), BF16),
        scratch_shapes=[pltpu.VMEM((TQA, VP), F32)],
    )(q, k, v)

    out = pl.pallas_call(
        _moe_body,
        grid=(E, HB),
        in_specs=[
            pl.BlockSpec((H, S, DH), lambda e, hh: (0, 0, 0)),
            pl.BlockSpec((H * DH, DIM), lambda e, hh: (0, 0)),
            pl.BlockSpec((DIM, GPAD), lambda e, hh: (0, 0)),
            pl.BlockSpec((1, DIM, KH), lambda e, hh: (e, 0, hh)),
            pl.BlockSpec((1, 1, KH), lambda e, hh: (e, 0, hh)),
            pl.BlockSpec((1, KH, DIM), lambda e, hh: (e, hh, 0)),
            pl.BlockSpec((1, 1, DIM), lambda e, hh: (e, 0, 0)),
        ],
        out_specs=pl.BlockSpec((S, DIM), lambda e, hh: (0, 0)),
        out_shape=jax.ShapeDtypeStruct((S, DIM), F32),
        scratch_shapes=[
            pltpu.VMEM((S, DIM), BF16),
            pltpu.VMEM((S, GPAD), F32),
        ],
    )(attn, wo_b, wg_pad, W1, b1_3d, W2, b2_3d)

    return out.reshape(B, S, DIM)


# speedup vs baseline: 1.2257x; 1.2257x over previous
import jax
import jax.numpy as jnp
from jax.experimental import pallas as pl
from jax.experimental.pallas import tpu as pltpu

B, S, DIM = 1, 2048, 1024
DH, H = 64, 16
E, HID = 8, 4096
EPS = 1e-6
SCALE = DH ** (-0.5)

TQ = 256
TQA = 256
TKA = 256
NQ = S // TQA
NK = S // TKA
VP = 128
KH = 1024
KHS = KH // 2
HB = HID // KH
GPAD = 128

F32 = jnp.float32
BF16 = jnp.bfloat16


def _bdot(a, b):
    return jax.lax.dot_general(a.astype(BF16), b.astype(BF16),
                               (((1,), (0,)), ((), ())),
                               preferred_element_type=F32)


def _qkv_body(x_ref, wq_ref, wk_ref, wv_ref, gq_ref, gk_ref, q_ref, k_ref, v_ref):
    xb = x_ref[...].astype(BF16)
    q = jnp.dot(xb, wq_ref[...], preferred_element_type=F32)
    k = jnp.dot(xb, wk_ref[...], preferred_element_type=F32)
    v = jnp.dot(xb, wv_ref[...], preferred_element_type=F32)
    gq = gq_ref[...]
    gk = gk_ref[...]
    ones_col = (jax.lax.broadcasted_iota(jnp.int32, (TQ, VP - DH), 1) == 0)
    pad = jnp.where(ones_col, 1.0, 0.0).astype(BF16)
    for hh in range(H):
        sl = slice(hh * DH, (hh + 1) * DH)
        qh = q[:, sl]
        kh = k[:, sl]
        qms = jnp.mean(qh * qh, axis=1, keepdims=True)
        kms = jnp.mean(kh * kh, axis=1, keepdims=True)
        q_ref[hh] = (qh * (jax.lax.rsqrt(qms + EPS) * SCALE) * gq[:, sl]).astype(BF16)
        k_ref[hh] = (kh * jax.lax.rsqrt(kms + EPS) * gk[:, sl]).astype(BF16)
        v_ref[hh] = jnp.concatenate([v[:, sl].astype(BF16), pad], axis=1)


def _attn_body(q_ref, k_ref, v_ref, o_ref):
    qi = pl.program_id(1)
    q = q_ref[0]

    def emit(nck):
        ncols = nck * TKA
        kc = k_ref[0, :ncols, :]
        vc = v_ref[0, :ncols, :]
        s = jax.lax.dot_general(q, kc, (((1,), (1,)), ((), ())),
                                preferred_element_type=F32)
        p = jnp.exp(s)
        rows = qi * TQA + jax.lax.broadcasted_iota(jnp.int32, (TQA, ncols), 0)
        cols = jax.lax.broadcasted_iota(jnp.int32, (TQA, ncols), 1)
        p = jnp.where(rows >= cols, p, 0.0)
        acc = jnp.dot(p.astype(BF16), vc, preferred_element_type=F32)
        colsa = jax.lax.broadcasted_iota(jnp.int32, acc.shape, 1)
        denom = jnp.sum(jnp.where(colsa == DH, acc, 0.0), axis=1, keepdims=True)
        o_ref[0] = (acc[:, :DH] / denom).astype(BF16)

    for b, nck in enumerate((2, 4, 6, 8)):
        @pl.when((qi >= b * 2) & (qi < b * 2 + 2))
        def _branch(nck=nck):
            emit(nck)


def _moe_body(a_ref, wo_ref, wg_ref, w1_ref, b1_ref, w2_ref, b2_ref,
              out_ref, o_sc, g_sc):
    e = pl.program_id(0)
    h = pl.program_id(1)

    @pl.when((e == 0) & (h == 0))
    def _prologue():
        ob = jnp.zeros((S, DIM), F32)
        for hh in range(H):
            ob = ob + jnp.dot(a_ref[hh], wo_ref[hh * DH:(hh + 1) * DH, :],
                              preferred_element_type=F32)
        o_sc[...] = ob.astype(BF16)
        gl = jnp.dot(o_sc[...], wg_ref[...], preferred_element_type=F32)
        cols = jax.lax.broadcasted_iota(jnp.int32, gl.shape, 1)
        gl = jnp.where(cols < E, gl, -jnp.inf)
        m = jnp.max(gl, axis=1, keepdims=True)
        p = jnp.exp(gl - m)
        g_sc[...] = p / jnp.sum(p, axis=1, keepdims=True)
        out_ref[...] = jnp.zeros_like(out_ref)

    ob = o_sc[...]
    hb0 = _bdot(ob, w1_ref[0, :, :KHS]) + b1_ref[0, :, :KHS]
    hb1 = _bdot(ob, w1_ref[0, :, KHS:]) + b1_ref[0, :, KHS:]
    contrib = (_bdot(jax.nn.gelu(hb0), w2_ref[0, :KHS, :])
               + _bdot(jax.nn.gelu(hb1), w2_ref[0, KHS:, :]))
    g = g_sc[...]
    cols = jax.lax.broadcasted_iota(jnp.int32, g.shape, 1)
    ge = jnp.sum(jnp.where(cols == e, g, 0.0), axis=1, keepdims=True)
    acc = out_ref[...] + ge * contrib

    @pl.when(h == HB - 1)
    def _bias():
        out_ref[...] = acc + ge * b2_ref[0]

    @pl.when(h != HB - 1)
    def _noacc():
        out_ref[...] = acc


def kernel(x, Wq, Wk, Wv, Wo, gq, gk, Wg, W1, b1, W2, b2):
    xs = x.reshape(S, DIM)
    gq_t = jnp.tile(gq, H).reshape(1, H * DH)
    gk_t = jnp.tile(gk, H).reshape(1, H * DH)
    wg_pad = jnp.zeros((DIM, GPAD), BF16).at[:, :E].set(Wg.astype(BF16))
    b1_3d = b1.reshape(E, 1, HID)
    b2_3d = b2.reshape(E, 1, DIM)
    wq_b, wk_b, wv_b = Wq.astype(BF16), Wk.astype(BF16), Wv.astype(BF16)
    wo_b = Wo.astype(BF16)

    q, k, v = pl.pallas_call(
        _qkv_body,
        grid=(S // TQ,),
        in_specs=[
            pl.BlockSpec((TQ, DIM), lambda i: (i, 0)),
            pl.BlockSpec((DIM, H * DH), lambda i: (0, 0)),
            pl.BlockSpec((DIM, H * DH), lambda i: (0, 0)),
            pl.BlockSpec((DIM, H * DH), lambda i: (0, 0)),
            pl.BlockSpec((1, H * DH), lambda i: (0, 0)),
            pl.BlockSpec((1, H * DH), lambda i: (0, 0)),
        ],
        out_specs=[
            pl.BlockSpec((H, TQ, DH), lambda i: (0, i, 0)),
            pl.BlockSpec((H, TQ, DH), lambda i: (0, i, 0)),
            pl.BlockSpec((H, TQ, VP), lambda i: (0, i, 0)),
        ],
        out_shape=[
            jax.ShapeDtypeStruct((H, S, DH), BF16),
            jax.ShapeDtypeStruct((H, S, DH), BF16),
            jax.ShapeDtypeStruct((H, S, VP), BF16),
        ],
    )(xs, wq_b, wk_b, wv_b, gq_t, gk_t)

    attn = pl.pallas_call(
        _attn_body,
        grid=(H, NQ),
        in_specs=[
            pl.BlockSpec((1, TQA, DH), lambda hh, i: (hh, i, 0)),
            pl.BlockSpec((1, S, DH), lambda hh, i: (hh, 0, 0)),
            pl.BlockSpec((1, S, VP), lambda hh, i: (hh, 0, 0)),
        ],
        out_specs=pl.BlockSpec((1, TQA, DH), lambda hh, i: (hh, i, 0)),
        out_shape=jax.ShapeDtypeStruct((H, S, DH), BF16),
    )(q, k, v)

    out = pl.pallas_call(
        _moe_body,
        grid=(E, HB),
        in_specs=[
            pl.BlockSpec((H, S, DH), lambda e, hh: (0, 0, 0)),
            pl.BlockSpec((H * DH, DIM), lambda e, hh: (0, 0)),
            pl.BlockSpec((DIM, GPAD), lambda e, hh: (0, 0)),
            pl.BlockSpec((1, DIM, KH), lambda e, hh: (e, 0, hh)),
            pl.BlockSpec((1, 1, KH), lambda e, hh: (e, 0, hh)),
            pl.BlockSpec((1, KH, DIM), lambda e, hh: (e, hh, 0)),
            pl.BlockSpec((1, 1, DIM), lambda e, hh: (e, 0, 0)),
        ],
        out_specs=pl.BlockSpec((S, DIM), lambda e, hh: (0, 0)),
        out_shape=jax.ShapeDtypeStruct((S, DIM), F32),
        scratch_shapes=[
            pltpu.VMEM((S, DIM), BF16),
            pltpu.VMEM((S, GPAD), F32),
        ],
    )(attn, wo_b, wg_pad, W1, b1_3d, W2, b2_3d)

    return out.reshape(B, S, DIM)


# speedup vs baseline: 1.2520x; 1.0214x over previous
import jax
import jax.numpy as jnp
from jax.experimental import pallas as pl
from jax.experimental.pallas import tpu as pltpu

B, S, DIM = 1, 2048, 1024
DH, H = 64, 16
E, HID = 8, 4096
EPS = 1e-6
SCALE = DH ** (-0.5)

TQ = 256
TQA = 256
TKA = 256
NQ = S // TQA
NK = S // TKA
VP = 128
KH = 1024
KHS = KH // 2
HB = HID // KH
GPAD = 128

F32 = jnp.float32
BF16 = jnp.bfloat16


def _bdot(a, b):
    return jax.lax.dot_general(a.astype(BF16), b.astype(BF16),
                               (((1,), (0,)), ((), ())),
                               preferred_element_type=F32)


def _qkv_body(x_ref, wq_ref, wk_ref, wv_ref, gq_ref, gk_ref, dred_ref, dexp_ref,
              q_ref, k_ref, v_ref):
    xb = x_ref[...].astype(BF16)
    q = jnp.dot(xb, wq_ref[...], preferred_element_type=F32)
    k = jnp.dot(xb, wk_ref[...], preferred_element_type=F32)
    v = jnp.dot(xb, wv_ref[...], preferred_element_type=F32)
    gq = gq_ref[...]
    gk = gk_ref[...]
    dred = dred_ref[...]
    dexp = dexp_ref[...]
    qrs = jax.lax.rsqrt(_bdot(q * q, dred) + EPS)
    krs = jax.lax.rsqrt(_bdot(k * k, dred) + EPS)
    qn = (q * _bdot(qrs, dexp) * (gq * SCALE)).astype(BF16)
    kn = (k * _bdot(krs, dexp) * gk).astype(BF16)
    ones_col = (jax.lax.broadcasted_iota(jnp.int32, (TQ, VP - DH), 1) == 0)
    pad = jnp.where(ones_col, 1.0, 0.0).astype(BF16)
    for hh in range(H):
        sl = slice(hh * DH, (hh + 1) * DH)
        q_ref[hh] = qn[:, sl]
        k_ref[hh] = kn[:, sl]
        v_ref[hh] = jnp.concatenate([v[:, sl].astype(BF16), pad], axis=1)


def _attn_body(q_ref, k_ref, v_ref, o_ref):
    qi = pl.program_id(1)
    q = q_ref[0]

    def emit(nck):
        ncols = nck * TKA
        kc = k_ref[0, :ncols, :]
        vc = v_ref[0, :ncols, :]
        s = jax.lax.dot_general(q, kc, (((1,), (1,)), ((), ())),
                                preferred_element_type=F32)
        p = jnp.exp(s)
        rows = qi * TQA + jax.lax.broadcasted_iota(jnp.int32, (TQA, ncols), 0)
        cols = jax.lax.broadcasted_iota(jnp.int32, (TQA, ncols), 1)
        p = jnp.where(rows >= cols, p, 0.0)
        acc = jnp.dot(p.astype(BF16), vc, preferred_element_type=F32)
        colsa = jax.lax.broadcasted_iota(jnp.int32, acc.shape, 1)
        denom = jnp.sum(jnp.where(colsa == DH, acc, 0.0), axis=1, keepdims=True)
        o_ref[0] = (acc[:, :DH] / denom).astype(BF16)

    for b in range(NQ):
        @pl.when(qi == b)
        def _branch(nck=b + 1):
            emit(nck)


def _moe_body(a_ref, wo_ref, wg_ref, w1_ref, b1_ref, w2_ref, b2_ref,
              out_ref, o_sc, g_sc):
    e = pl.program_id(0)
    h = pl.program_id(1)

    @pl.when((e == 0) & (h == 0))
    def _prologue():
        ob = jnp.zeros((S, DIM), F32)
        for hh in range(H):
            ob = ob + jnp.dot(a_ref[hh], wo_ref[hh * DH:(hh + 1) * DH, :],
                              preferred_element_type=F32)
        o_sc[...] = ob.astype(BF16)
        gl = jnp.dot(o_sc[...], wg_ref[...], preferred_element_type=F32)
        cols = jax.lax.broadcasted_iota(jnp.int32, gl.shape, 1)
        gl = jnp.where(cols < E, gl, -jnp.inf)
        m = jnp.max(gl, axis=1, keepdims=True)
        p = jnp.exp(gl - m)
        g_sc[...] = p / jnp.sum(p, axis=1, keepdims=True)
        out_ref[...] = jnp.zeros_like(out_ref)

    ob = o_sc[...]
    hb0 = _bdot(ob, w1_ref[0, :, :KHS]) + b1_ref[0, :, :KHS]
    hb1 = _bdot(ob, w1_ref[0, :, KHS:]) + b1_ref[0, :, KHS:]
    contrib = (_bdot(jax.nn.gelu(hb0), w2_ref[0, :KHS, :])
               + _bdot(jax.nn.gelu(hb1), w2_ref[0, KHS:, :]))
    g = g_sc[...]
    cols = jax.lax.broadcasted_iota(jnp.int32, g.shape, 1)
    ge = jnp.sum(jnp.where(cols == e, g, 0.0), axis=1, keepdims=True)
    acc = out_ref[...] + ge * contrib

    @pl.when(h == HB - 1)
    def _bias():
        out_ref[...] = acc + ge * b2_ref[0]

    @pl.when(h != HB - 1)
    def _noacc():
        out_ref[...] = acc


def kernel(x, Wq, Wk, Wv, Wo, gq, gk, Wg, W1, b1, W2, b2):
    xs = x.reshape(S, DIM)
    gq_t = jnp.tile(gq, H).reshape(1, H * DH)
    gk_t = jnp.tile(gk, H).reshape(1, H * DH)
    wg_pad = jnp.zeros((DIM, GPAD), BF16).at[:, :E].set(Wg.astype(BF16))
    b1_3d = b1.reshape(E, 1, HID)
    b2_3d = b2.reshape(E, 1, DIM)
    wq_b, wk_b, wv_b = Wq.astype(BF16), Wk.astype(BF16), Wv.astype(BF16)
    wo_b = Wo.astype(BF16)
    head_of_col = jnp.arange(H * DH, dtype=jnp.int32) // DH
    hcols = jnp.arange(GPAD, dtype=jnp.int32)
    memb = (head_of_col[:, None] == hcols[None, :])
    dred = jnp.where(memb, 1.0 / DH, 0.0).astype(BF16)
    dexp = jnp.where(memb.T, 1.0, 0.0).astype(BF16)

    q, k, v = pl.pallas_call(
        _qkv_body,
        grid=(S // TQ,),
        in_specs=[
            pl.BlockSpec((TQ, DIM), lambda i: (i, 0)),
            pl.BlockSpec((DIM, H * DH), lambda i: (0, 0)),
            pl.BlockSpec((DIM, H * DH), lambda i: (0, 0)),
            pl.BlockSpec((DIM, H * DH), lambda i: (0, 0)),
            pl.BlockSpec((1, H * DH), lambda i: (0, 0)),
            pl.BlockSpec((1, H * DH), lambda i: (0, 0)),
            pl.BlockSpec((H * DH, GPAD), lambda i: (0, 0)),
            pl.BlockSpec((GPAD, H * DH), lambda i: (0, 0)),
        ],
        out_specs=[
            pl.BlockSpec((H, TQ, DH), lambda i: (0, i, 0)),
            pl.BlockSpec((H, TQ, DH), lambda i: (0, i, 0)),
            pl.BlockSpec((H, TQ, VP), lambda i: (0, i, 0)),
        ],
        out_shape=[
            jax.ShapeDtypeStruct((H, S, DH), BF16),
            jax.ShapeDtypeStruct((H, S, DH), BF16),
            jax.ShapeDtypeStruct((H, S, VP), BF16),
        ],
    )(xs, wq_b, wk_b, wv_b, gq_t, gk_t, dred, dexp)

    attn = pl.pallas_call(
        _attn_body,
        grid=(H, NQ),
        in_specs=[
            pl.BlockSpec((1, TQA, DH), lambda hh, i: (hh, i, 0)),
            pl.BlockSpec((1, S, DH), lambda hh, i: (hh, 0, 0)),
            pl.BlockSpec((1, S, VP), lambda hh, i: (hh, 0, 0)),
        ],
        out_specs=pl.BlockSpec((1, TQA, DH), lambda hh, i: (hh, i, 0)),
        out_shape=jax.ShapeDtypeStruct((H, S, DH), BF16),
    )(q, k, v)

    out = pl.pallas_call(
        _moe_body,
        grid=(E, HB),
        in_specs=[
            pl.BlockSpec((H, S, DH), lambda e, hh: (0, 0, 0)),
            pl.BlockSpec((H * DH, DIM), lambda e, hh: (0, 0)),
            pl.BlockSpec((DIM, GPAD), lambda e, hh: (0, 0)),
            pl.BlockSpec((1, DIM, KH), lambda e, hh: (e, 0, hh)),
            pl.BlockSpec((1, 1, KH), lambda e, hh: (e, 0, hh)),
            pl.BlockSpec((1, KH, DIM), lambda e, hh: (e, hh, 0)),
            pl.BlockSpec((1, 1, DIM), lambda e, hh: (e, 0, 0)),
        ],
        out_specs=pl.BlockSpec((S, DIM), lambda e, hh: (0, 0)),
        out_shape=jax.ShapeDtypeStruct((S, DIM), F32),
        scratch_shapes=[
            pltpu.VMEM((S, DIM), BF16),
            pltpu.VMEM((S, GPAD), F32),
        ],
    )(attn, wo_b, wg_pad, W1, b1_3d, W2, b2_3d)

    return out.reshape(B, S, DIM)


# speedup vs baseline: 1.2527x; 1.0006x over previous
import jax
import jax.numpy as jnp
from jax.experimental import pallas as pl
from jax.experimental.pallas import tpu as pltpu

B, S, DIM = 1, 2048, 1024
DH, H = 64, 16
E, HID = 8, 4096
EPS = 1e-6
SCALE = DH ** (-0.5)

TQ = 256
TQA = 256
TKA = 256
NQ = S // TQA
NK = S // TKA
VP = 128
KH = 1024
KHS = 512
HB = HID // KH
GPAD = 128

F32 = jnp.float32
BF16 = jnp.bfloat16


def _bdot(a, b):
    return jax.lax.dot_general(a.astype(BF16), b.astype(BF16),
                               (((1,), (0,)), ((), ())),
                               preferred_element_type=F32)


def _qkv_body(x_ref, wq_ref, wk_ref, wv_ref, gq_ref, gk_ref, dred_ref, dexp_ref,
              q_ref, k_ref, v_ref):
    xb = x_ref[...].astype(BF16)
    q = jnp.dot(xb, wq_ref[...], preferred_element_type=F32)
    k = jnp.dot(xb, wk_ref[...], preferred_element_type=F32)
    v = jnp.dot(xb, wv_ref[...], preferred_element_type=F32)
    gq = gq_ref[...]
    gk = gk_ref[...]
    dred = dred_ref[...]
    dexp = dexp_ref[...]
    qrs = jax.lax.rsqrt(_bdot(q * q, dred) + EPS)
    krs = jax.lax.rsqrt(_bdot(k * k, dred) + EPS)
    qn = (q * _bdot(qrs, dexp) * (gq * SCALE)).astype(BF16)
    kn = (k * _bdot(krs, dexp) * gk).astype(BF16)
    ones_col = (jax.lax.broadcasted_iota(jnp.int32, (TQ, VP - DH), 1) == 0)
    pad = jnp.where(ones_col, 1.0, 0.0).astype(BF16)
    for hh in range(H):
        sl = slice(hh * DH, (hh + 1) * DH)
        q_ref[hh] = qn[:, sl]
        k_ref[hh] = kn[:, sl]
        v_ref[hh] = jnp.concatenate([v[:, sl].astype(BF16), pad], axis=1)


def _attn_body(q_ref, k_ref, v_ref, o_ref):
    qi = pl.program_id(1)
    q = q_ref[0]

    def emit(nck):
        ncols = nck * TKA
        kc = k_ref[0, :ncols, :]
        vc = v_ref[0, :ncols, :]
        s = jax.lax.dot_general(q, kc, (((1,), (1,)), ((), ())),
                                preferred_element_type=F32)
        p = jnp.exp(s)
        rows = qi * TQA + jax.lax.broadcasted_iota(jnp.int32, (TQA, ncols), 0)
        cols = jax.lax.broadcasted_iota(jnp.int32, (TQA, ncols), 1)
        p = jnp.where(rows >= cols, p, 0.0)
        acc = jnp.dot(p.astype(BF16), vc, preferred_element_type=F32)
        colsa = jax.lax.broadcasted_iota(jnp.int32, acc.shape, 1)
        denom = jnp.sum(jnp.where(colsa == DH, acc, 0.0), axis=1, keepdims=True)
        o_ref[0] = (acc[:, :DH] / denom).astype(BF16)

    for b in range(NQ):
        @pl.when(qi == b)
        def _branch(nck=b + 1):
            emit(nck)


def _moe_body(a_ref, wo_ref, wg_ref, w1_ref, b1_ref, w2_ref, b2_ref,
              out_ref, o_sc, g_sc):
    e = pl.program_id(0)
    h = pl.program_id(1)

    @pl.when((e == 0) & (h == 0))
    def _prologue():
        ob = jnp.zeros((S, DIM), F32)
        for hh in range(H):
            ob = ob + jnp.dot(a_ref[hh], wo_ref[hh * DH:(hh + 1) * DH, :],
                              preferred_element_type=F32)
        o_sc[...] = ob.astype(BF16)
        gl = jnp.dot(o_sc[...], wg_ref[...], preferred_element_type=F32)
        cols = jax.lax.broadcasted_iota(jnp.int32, gl.shape, 1)
        gl = jnp.where(cols < E, gl, -jnp.inf)
        m = jnp.max(gl, axis=1, keepdims=True)
        p = jnp.exp(gl - m)
        g_sc[...] = p / jnp.sum(p, axis=1, keepdims=True)
        out_ref[...] = jnp.zeros_like(out_ref)

    ob = o_sc[...]
    contrib = jnp.zeros((S, DIM), F32)
    for c in range(KH // KHS):
        sl = slice(c * KHS, (c + 1) * KHS)
        hbc = _bdot(ob, w1_ref[0, :, sl]) + b1_ref[0, :, sl]
        contrib = contrib + _bdot(jax.nn.gelu(hbc), w2_ref[0, sl, :])
    g = g_sc[...]
    cols = jax.lax.broadcasted_iota(jnp.int32, g.shape, 1)
    ge = jnp.sum(jnp.where(cols == e, g, 0.0), axis=1, keepdims=True)
    acc = out_ref[...] + ge * contrib

    @pl.when(h == HB - 1)
    def _bias():
        out_ref[...] = acc + ge * b2_ref[0]

    @pl.when(h != HB - 1)
    def _noacc():
        out_ref[...] = acc


def kernel(x, Wq, Wk, Wv, Wo, gq, gk, Wg, W1, b1, W2, b2):
    xs = x.reshape(S, DIM)
    gq_t = jnp.tile(gq, H).reshape(1, H * DH)
    gk_t = jnp.tile(gk, H).reshape(1, H * DH)
    wg_pad = jnp.zeros((DIM, GPAD), BF16).at[:, :E].set(Wg.astype(BF16))
    b1_3d = b1.reshape(E, 1, HID)
    b2_3d = b2.reshape(E, 1, DIM)
    wq_b, wk_b, wv_b = Wq.astype(BF16), Wk.astype(BF16), Wv.astype(BF16)
    wo_b = Wo.astype(BF16)
    head_of_col = jnp.arange(H * DH, dtype=jnp.int32) // DH
    hcols = jnp.arange(GPAD, dtype=jnp.int32)
    memb = (head_of_col[:, None] == hcols[None, :])
    dred = jnp.where(memb, 1.0 / DH, 0.0).astype(BF16)
    dexp = jnp.where(memb.T, 1.0, 0.0).astype(BF16)

    q, k, v = pl.pallas_call(
        _qkv_body,
        grid=(S // TQ,),
        in_specs=[
            pl.BlockSpec((TQ, DIM), lambda i: (i, 0)),
            pl.BlockSpec((DIM, H * DH), lambda i: (0, 0)),
            pl.BlockSpec((DIM, H * DH), lambda i: (0, 0)),
            pl.BlockSpec((DIM, H * DH), lambda i: (0, 0)),
            pl.BlockSpec((1, H * DH), lambda i: (0, 0)),
            pl.BlockSpec((1, H * DH), lambda i: (0, 0)),
            pl.BlockSpec((H * DH, GPAD), lambda i: (0, 0)),
            pl.BlockSpec((GPAD, H * DH), lambda i: (0, 0)),
        ],
        out_specs=[
            pl.BlockSpec((H, TQ, DH), lambda i: (0, i, 0)),
            pl.BlockSpec((H, TQ, DH), lambda i: (0, i, 0)),
            pl.BlockSpec((H, TQ, VP), lambda i: (0, i, 0)),
        ],
        out_shape=[
            jax.ShapeDtypeStruct((H, S, DH), BF16),
            jax.ShapeDtypeStruct((H, S, DH), BF16),
            jax.ShapeDtypeStruct((H, S, VP), BF16),
        ],
    )(xs, wq_b, wk_b, wv_b, gq_t, gk_t, dred, dexp)

    attn = pl.pallas_call(
        _attn_body,
        grid=(H, NQ),
        in_specs=[
            pl.BlockSpec((1, TQA, DH), lambda hh, i: (hh, i, 0)),
            pl.BlockSpec((1, S, DH), lambda hh, i: (hh, 0, 0)),
            pl.BlockSpec((1, S, VP), lambda hh, i: (hh, 0, 0)),
        ],
        out_specs=pl.BlockSpec((1, TQA, DH), lambda hh, i: (hh, i, 0)),
        out_shape=jax.ShapeDtypeStruct((H, S, DH), BF16),
    )(q, k, v)

    out = pl.pallas_call(
        _moe_body,
        grid=(E, HB),
        in_specs=[
            pl.BlockSpec((H, S, DH), lambda e, hh: (0, 0, 0)),
            pl.BlockSpec((H * DH, DIM), lambda e, hh: (0, 0)),
            pl.BlockSpec((DIM, GPAD), lambda e, hh: (0, 0)),
            pl.BlockSpec((1, DIM, KH), lambda e, hh: (e, 0, hh)),
            pl.BlockSpec((1, 1, KH), lambda e, hh: (e, 0, hh)),
            pl.BlockSpec((1, KH, DIM), lambda e, hh: (e, hh, 0)),
            pl.BlockSpec((1, 1, DIM), lambda e, hh: (e, 0, 0)),
        ],
        out_specs=pl.BlockSpec((S, DIM), lambda e, hh: (0, 0)),
        out_shape=jax.ShapeDtypeStruct((S, DIM), F32),
        scratch_shapes=[
            pltpu.VMEM((S, DIM), BF16),
            pltpu.VMEM((S, GPAD), F32),
        ],
    )(attn, wo_b, wg_pad, W1, b1_3d, W2, b2_3d)

    return out.reshape(B, S, DIM)
